# Initial kernel scaffold; baseline (speedup 1.0000x reference)
import jax
import jax.numpy as jnp
from jax.experimental import pallas as pl


def kernel(inputs, emb, W1, b1, W2, b2):
    raise NotImplementedError("write your pallas kernel here")


# speedup vs baseline: 4.8491x; 4.8491x over previous
import functools

import jax
import jax.numpy as jnp
from jax import lax
from jax.experimental import pallas as pl
from jax.experimental.pallas import tpu as pltpu
from jax.experimental.pallas import tpu_sc as plsc

VOCAB = 2000
EMB_DIM = 32
D1 = 256
D2 = 10
BATCH = 16384
SEQ = 200

NC = 2
NS = 16
NW = NC * NS

BS = BATCH * SEQ
PER_W = BS // NW
CHUNK = 2048
NIT = PER_W // CHUNK
GROUPS = CHUNK // 16


def _table_body(emb_ref, w1_ref, b1_ref, w2_ref, b2_ref, out_ref):
    h = jnp.dot(emb_ref[...], w1_ref[...], preferred_element_type=jnp.float32)
    h = jnp.maximum(h + b1_ref[...], 0.0)
    z = jnp.dot(h, w2_ref[...], preferred_element_type=jnp.float32) + b2_ref[...]
    z = z - jnp.max(z, axis=-1, keepdims=True)
    e = jnp.exp(z)
    out_ref[...] = e / jnp.sum(e, axis=-1, keepdims=True)


def _make_table(emb, W1, b1, W2, b2):
    return pl.pallas_call(
        _table_body,
        out_shape=jax.ShapeDtypeStruct((VOCAB, D2), jnp.float32),
    )(emb, W1, b1.reshape(1, D1), W2, b2.reshape(1, D2))


def _gather_body(table_hbm, idx_hbm, out_hbm, table_v, idx_v, out_v):
    wid = lax.axis_index("s") * NC + lax.axis_index("c")
    base = wid * PER_W
    pltpu.sync_copy(table_hbm, table_v)

    iota = lax.iota(jnp.int32, 16)
    scat0 = iota * D2

    def chunk_body(it, _):
        off = base + it * CHUNK
        pltpu.sync_copy(idx_hbm.at[pl.ds(off, CHUNK)], idx_v)

        def group_body(g, _):
            rows = idx_v[pl.ds(g * 16, 16)]
            r10 = rows * D2
            sbase = scat0 + g * (16 * D2)
            for d in range(D2):
                vals = plsc.load_gather(table_v, [r10 + d])
                plsc.store_scatter(out_v, [sbase + d], vals)
            return 0

        lax.fori_loop(0, GROUPS, group_body, 0, unroll=False)
        pltpu.sync_copy(out_v, out_hbm.at[pl.ds(off * D2, CHUNK * D2)])
        return 0

    lax.fori_loop(0, NIT, chunk_body, 0, unroll=False)


@functools.cache
def _make_gather():
    return functools.partial(
        pl.kernel,
        out_type=jax.ShapeDtypeStruct((BS * D2,), jnp.float32),
        mesh=plsc.VectorSubcoreMesh(
            core_axis_name="c", subcore_axis_name="s", num_cores=NC, num_subcores=NS
        ),
        scratch_types=[
            pltpu.VMEM((VOCAB * D2,), jnp.float32),
            pltpu.VMEM((CHUNK,), jnp.int32),
            pltpu.VMEM((CHUNK * D2,), jnp.float32),
        ],
        compiler_params=pltpu.CompilerParams(needs_layout_passes=False),
    )(_gather_body)


def kernel(inputs, emb, W1, b1, W2, b2):
    table = _make_table(emb, W1, b1, W2, b2)
    idx = inputs.reshape(-1).astype(jnp.int32)
    out = _make_gather()(table.reshape(-1), idx)
    return out.reshape(BATCH, SEQ, D2)


# speedup vs baseline: 145.4545x; 29.9959x over previous
import functools

import jax
import jax.numpy as jnp
from jax import lax
from jax.experimental import pallas as pl
from jax.experimental.pallas import tpu as pltpu
from jax.experimental.pallas import tpu_sc as plsc

VOCAB = 2000
EMB_DIM = 32
D1 = 256
D2 = 10
BATCH = 16384
SEQ = 200

NC = 2
NS = 16
NW = NC * NS

BS = BATCH * SEQ
ST = SEQ // 8
BB = BATCH // 128
BB_W = BB // NW
PIECE = BB_W * 8 * 128
GROUPS = PIECE // 16


def _table_body(emb_ref, w1_ref, b1_ref, w2_ref, b2_ref, out_ref):
    h = jnp.dot(emb_ref[...], w1_ref[...], preferred_element_type=jnp.float32)
    h = jnp.maximum(h + b1_ref[...], 0.0)
    z = jnp.dot(h, w2_ref[...], preferred_element_type=jnp.float32) + b2_ref[...]
    z = z - jnp.max(z, axis=-1, keepdims=True)
    e = jnp.exp(z)
    out_ref[...] = e / jnp.sum(e, axis=-1, keepdims=True)


def _make_table(emb, W1, b1, W2, b2):
    return pl.pallas_call(
        _table_body,
        out_shape=jax.ShapeDtypeStruct((VOCAB, D2), jnp.float32),
    )(emb, W1, b1.reshape(1, D1), W2, b2.reshape(1, D2))


def _compute_piece(table_v, idx_ref, out_ref):
    @plsc.parallel_loop(0, PIECE, step=16, unroll=4)
    def group_body(c):
        rows = idx_ref[pl.ds(c, 16)]
        r10 = rows * D2
        for d in range(D2):
            vals = plsc.load_gather(table_v, [r10 + d])
            out_ref[pl.ds(d * PIECE + c, 16)] = vals


def _gather_body(
    table_hbm, idx_hbm, out_hbm,
    table_v, idx_a, idx_b, out_a, out_b,
    sem_t, sem_i0, sem_i1, sem_o0, sem_o1,
):
    wid = lax.axis_index("s") * NC + lax.axis_index("c")

    def idx_off(st):
        return (st * BB + BB_W * wid) * 1024

    def out_off(d, st):
        return ((d * ST + st) * BB + BB_W * wid) * 1024

    dt = pltpu.async_copy(table_hbm, table_v, sem_t)
    idx_bufs = (idx_a, idx_b)
    out_bufs = (out_a, out_b)
    isems = (sem_i0, sem_i1)
    osems = (sem_o0, sem_o1)
    idesc = [
        pltpu.async_copy(idx_hbm.at[pl.ds(idx_off(0), PIECE)], idx_a, sem_i0),
        pltpu.async_copy(idx_hbm.at[pl.ds(idx_off(1), PIECE)], idx_b, sem_i1),
    ]
    odesc = [None, None]
    dt.wait()
    for st in range(ST):
        slot = st & 1
        idesc[slot].wait()
        if odesc[slot] is not None:
            for dd in odesc[slot]:
                dd.wait()
        _compute_piece(table_v, idx_bufs[slot], out_bufs[slot])
        if st + 2 < ST:
            idesc[slot] = pltpu.async_copy(
                idx_hbm.at[pl.ds(idx_off(st + 2), PIECE)],
                idx_bufs[slot],
                isems[slot],
            )
        odesc[slot] = [
            pltpu.async_copy(
                out_bufs[slot].at[pl.ds(d * PIECE, PIECE)],
                out_hbm.at[pl.ds(out_off(d, st), PIECE)],
                osems[slot],
            )
            for d in range(D2)
        ]
    for slot in (0, 1):
        for dd in odesc[slot]:
            dd.wait()


@functools.cache
def _make_gather():
    return functools.partial(
        pl.kernel,
        out_type=jax.ShapeDtypeStruct((BS * D2,), jnp.float32),
        mesh=plsc.VectorSubcoreMesh(
            core_axis_name="c", subcore_axis_name="s", num_cores=NC, num_subcores=NS
        ),
        scratch_types=[
            pltpu.VMEM((VOCAB * D2,), jnp.float32),
            pltpu.VMEM((PIECE,), jnp.int32),
            pltpu.VMEM((PIECE,), jnp.int32),
            pltpu.VMEM((D2 * PIECE,), jnp.float32),
            pltpu.VMEM((D2 * PIECE,), jnp.float32),
            pltpu.SemaphoreType.DMA,
            pltpu.SemaphoreType.DMA,
            pltpu.SemaphoreType.DMA,
            pltpu.SemaphoreType.DMA,
            pltpu.SemaphoreType.DMA,
        ],
        compiler_params=pltpu.CompilerParams(needs_layout_passes=False),
    )(_gather_body)


def kernel(inputs, emb, W1, b1, W2, b2):
    table = _make_table(emb, W1, b1, W2, b2)
    idx = (
        inputs.astype(jnp.int32)
        .reshape(128, 128, ST, 8)
        .transpose(2, 0, 3, 1)
        .reshape(-1)
    )
    out = _make_gather()(table.reshape(-1), idx)
    return (
        out.reshape(D2, ST, BB, 8, 128)
        .transpose(2, 4, 1, 3, 0)
        .reshape(BATCH, SEQ, D2)
    )
